# baseline (speedup 1.0000x reference)
import functools

import jax
import jax.numpy as jnp
from jax import lax
from jax.experimental import pallas as pl
from jax.experimental.pallas import tpu as pltpu
from jax.experimental.pallas import tpu_sc as plsc

N = 10000
D = 256
E = 160000
R = 16
B = 8
S = D // B
HALF = D // 2
NC = 2
NS = 16
CH = 128
EPT = 10240
STAGES = 2
CPS = EPT // (STAGES * CH)
EPAD = NS * EPT
NACC = N + 16
RPT = 640
TN_A = 1000
TN_C = 1000


def _ytab_body(x_ref, w_ref, out_ref, bdw):
    bdw[...] = jnp.zeros((D, D), jnp.float32)
    for b in range(B):
        bdw[b * S:(b + 1) * S, b * S:(b + 1) * S] = w_ref[0, b]
    y = jnp.dot(x_ref[...], bdw[...], preferred_element_type=jnp.float32)
    out_ref[0, 0] = y[:, :HALF]
    out_ref[1, 0] = y[:, HALF:]


def _build_ytab(x, W):
    out = pl.pallas_call(
        _ytab_body,
        grid=(N // TN_A, R),
        in_specs=[
            pl.BlockSpec((TN_A, D), lambda n, r: (n, 0)),
            pl.BlockSpec((1, B, S, S), lambda n, r: (r, 0, 0, 0)),
        ],
        out_specs=pl.BlockSpec((2, 1, TN_A, HALF), lambda n, r: (0, r, n, 0)),
        out_shape=jax.ShapeDtypeStruct((2, R, N, HALF), jnp.float32),
        scratch_shapes=[pltpu.VMEM((D, D), jnp.float32)],
    )(x, W)
    return out.reshape(2 * R * N, HALF)


def _scatter_body(ytab_hbm, gidx_hbm, dst_hbm, zeros_hbm, out_hbm,
                  gi_v, di_v, rows_v, acc, sg0, sg1, ss):
    sg = (sg0, sg1)
    c = lax.axis_index("c")
    s = lax.axis_index("s")
    zbase = pl.multiple_of(s * RPT, 128)

    @pl.when(s < NS - 1)
    def _():
        pltpu.sync_copy(zeros_hbm, acc.at[pl.ds(zbase, RPT)])

    @pl.when(s == NS - 1)
    def _():
        pltpu.sync_copy(zeros_hbm.at[pl.ds(0, 400)],
                        acc.at[pl.ds(zbase, 400)])

    plsc.subcore_barrier()
    grow = c * (EPAD // CH) + s * (EPT // CH)
    drow = s * (EPT // CH)

    for h in range(STAGES):
        goff = pl.multiple_of(grow + h * CPS, 8)
        doff = pl.multiple_of(drow + h * CPS, 8)
        pltpu.sync_copy(gidx_hbm.at[pl.ds(goff, CPS)], gi_v)
        pltpu.sync_copy(dst_hbm.at[pl.ds(doff, CPS)], di_v)
        pltpu.async_copy(ytab_hbm.at[gi_v.at[0]], rows_v.at[0], sg[0])
        pltpu.async_copy(ytab_hbm.at[gi_v.at[1]], rows_v.at[1], sg[1])

        def super_round(j, carry):
            for b in range(2):
                k = 2 * j + b
                pltpu.make_async_copy(ytab_hbm.at[gi_v.at[k]],
                                      rows_v.at[b], sg[b]).wait()
                pltpu.async_copy(rows_v.at[b], acc.at[di_v.at[k]], ss,
                                 add=True).wait()

                @pl.when(j < CPS // 2 - 1)
                def _():
                    pltpu.async_copy(ytab_hbm.at[gi_v.at[k + 2]],
                                     rows_v.at[b], sg[b])
            return carry

        lax.fori_loop(0, CPS // 2, super_round, 0)
    plsc.subcore_barrier()

    @pl.when(s < NS - 1)
    def _():
        pltpu.sync_copy(acc.at[pl.ds(zbase, RPT)],
                        out_hbm.at[c, pl.ds(zbase, RPT)])

    @pl.when(s == NS - 1)
    def _():
        pltpu.sync_copy(acc.at[pl.ds(zbase, 400)],
                        out_hbm.at[c, pl.ds(zbase, 400)])


def _segment_sum_sc(ytab, gidx_cat, dst_pad, zeros):
    mesh = plsc.VectorSubcoreMesh(
        core_axis_name="c", subcore_axis_name="s", num_cores=NC,
        num_subcores=NS)
    f = functools.partial(
        pl.kernel,
        out_type=jax.ShapeDtypeStruct((2, N, HALF), jnp.float32),
        mesh=mesh,
        scratch_types=[
            pltpu.VMEM((CPS, CH), jnp.int32),
            pltpu.VMEM((CPS, CH), jnp.int32),
            pltpu.VMEM((2, CH, HALF), jnp.float32),
            pltpu.VMEM_SHARED((NACC, HALF), jnp.float32),
            pltpu.SemaphoreType.DMA,
            pltpu.SemaphoreType.DMA,
            pltpu.SemaphoreType.DMA,
        ],
    )(_scatter_body)
    return f(ytab, gidx_cat, dst_pad, zeros)


def _selfloop_body(x_ref, lw_ref, out_ref):
    out_ref[...] = jnp.dot(x_ref[...], lw_ref[...],
                           preferred_element_type=jnp.float32)


def _selfloop(x, loop_weight):
    return pl.pallas_call(
        _selfloop_body,
        grid=(N // TN_C,),
        in_specs=[
            pl.BlockSpec((TN_C, D), lambda n: (n, 0)),
            pl.BlockSpec((D, D), lambda n: (0, 0)),
        ],
        out_specs=pl.BlockSpec((TN_C, D), lambda n: (n, 0)),
        out_shape=jax.ShapeDtypeStruct((N, D), jnp.float32),
    )(x, loop_weight)


def _final_body(agg_ref, sl_ref, p_ref, out_ref):
    agg = jnp.concatenate([agg_ref[0], agg_ref[1]], axis=-1)
    mu = jnp.mean(agg, axis=-1, keepdims=True)
    var = jnp.mean((agg - mu) * (agg - mu), axis=-1, keepdims=True)
    h = (agg - mu) * lax.rsqrt(var + 1e-5) * p_ref[0] + p_ref[1]
    out_ref[...] = h + sl_ref[...]


def _finalize(agg2, selfloop, params):
    return pl.pallas_call(
        _final_body,
        grid=(N // TN_C,),
        in_specs=[
            pl.BlockSpec((2, TN_C, HALF), lambda n: (0, n, 0)),
            pl.BlockSpec((TN_C, D), lambda n: (n, 0)),
            pl.BlockSpec((2, D), lambda n: (0, 0)),
        ],
        out_specs=pl.BlockSpec((TN_C, D), lambda n: (n, 0)),
        out_shape=jax.ShapeDtypeStruct((N, D), jnp.float32),
    )(agg2, selfloop, params)


def kernel(x, edge_index, rel, W, loop_weight, h_bias, ln_gamma, ln_beta):
    src = edge_index[0]
    dst = edge_index[1]
    gidx = rel * N + src
    g0 = jnp.pad(gidx, (0, EPAD - E))
    gidx_cat = jnp.concatenate([g0, g0 + R * N])
    gidx_cat = gidx_cat.reshape(2 * EPAD // CH, CH)
    dst_pad = jnp.pad(dst, (0, EPAD - E), constant_values=N)
    dst_pad = dst_pad.reshape(EPAD // CH, CH)
    zeros = jnp.zeros((RPT, HALF), jnp.float32)
    params = jnp.stack([ln_gamma, ln_beta + h_bias])

    ytab = _build_ytab(x, W)
    agg2 = _segment_sum_sc(ytab, gidx_cat, dst_pad, zeros)
    selfloop = _selfloop(x, loop_weight)
    return _finalize(agg2, selfloop, params)


# speedup vs baseline: 1.2221x; 1.2221x over previous
import functools

import jax
import jax.numpy as jnp
from jax import lax
from jax.experimental import pallas as pl
from jax.experimental.pallas import tpu as pltpu
from jax.experimental.pallas import tpu_sc as plsc

N = 10000
D = 256
E = 160000
R = 16
B = 8
S = D // B
HALF = D // 2
NC = 2
NS = 16
CH = 128
EPT = 10240
STAGES = 2
CPS = EPT // (STAGES * CH)
EPAD = NS * EPT
NACC = N + 16
RPT = 640
TN_A = 5000
TN_C = 1000


def _ytab_body(x_ref, w_ref, out_ref, bdw):
    bdw[...] = jnp.zeros((D, D), jnp.float32)
    for b in range(B):
        bdw[b * S:(b + 1) * S, b * S:(b + 1) * S] = w_ref[0, b]
    y = jnp.dot(x_ref[...], bdw[...], preferred_element_type=jnp.float32)
    out_ref[0, 0] = y[:, :HALF]
    out_ref[1, 0] = y[:, HALF:]


def _build_ytab(x, W):
    out = pl.pallas_call(
        _ytab_body,
        grid=(N // TN_A, R),
        in_specs=[
            pl.BlockSpec((TN_A, D), lambda n, r: (n, 0)),
            pl.BlockSpec((1, B, S, S), lambda n, r: (r, 0, 0, 0)),
        ],
        out_specs=pl.BlockSpec((2, 1, TN_A, HALF), lambda n, r: (0, r, n, 0)),
        out_shape=jax.ShapeDtypeStruct((2, R, N, HALF), jnp.float32),
        scratch_shapes=[pltpu.VMEM((D, D), jnp.float32)],
    )(x, W)
    return out.reshape(2 * R * N, HALF)


def _scatter_body(ytab_hbm, gidx_hbm, dst_hbm, zeros_hbm, out_hbm,
                  gi_v, di_v, rows_v, acc, sg0, sg1, ss):
    sg = (sg0, sg1)
    c = lax.axis_index("c")
    s = lax.axis_index("s")
    zbase = pl.multiple_of(s * RPT, 128)

    @pl.when(s < NS - 1)
    def _():
        pltpu.sync_copy(zeros_hbm, acc.at[pl.ds(zbase, RPT)])

    @pl.when(s == NS - 1)
    def _():
        pltpu.sync_copy(zeros_hbm.at[pl.ds(0, 400)],
                        acc.at[pl.ds(zbase, 400)])

    plsc.subcore_barrier()
    grow = c * (EPAD // CH) + s * (EPT // CH)
    drow = s * (EPT // CH)

    for h in range(STAGES):
        goff = pl.multiple_of(grow + h * CPS, 8)
        doff = pl.multiple_of(drow + h * CPS, 8)
        pltpu.sync_copy(gidx_hbm.at[pl.ds(goff, CPS)], gi_v)
        pltpu.sync_copy(dst_hbm.at[pl.ds(doff, CPS)], di_v)
        pltpu.async_copy(ytab_hbm.at[gi_v.at[0]], rows_v.at[0], sg[0])
        pltpu.async_copy(ytab_hbm.at[gi_v.at[1]], rows_v.at[1], sg[1])

        def super_round(j, carry):
            for b in range(2):
                k = 2 * j + b
                pltpu.make_async_copy(ytab_hbm.at[gi_v.at[k]],
                                      rows_v.at[b], sg[b]).wait()
                pltpu.async_copy(rows_v.at[b], acc.at[di_v.at[k]], ss,
                                 add=True).wait()

                @pl.when(j < CPS // 2 - 1)
                def _():
                    pltpu.async_copy(ytab_hbm.at[gi_v.at[k + 2]],
                                     rows_v.at[b], sg[b])
            return carry

        lax.fori_loop(0, CPS // 2, super_round, 0)
    plsc.subcore_barrier()

    @pl.when(s < NS - 1)
    def _():
        pltpu.sync_copy(acc.at[pl.ds(zbase, RPT)],
                        out_hbm.at[c, pl.ds(zbase, RPT)])

    @pl.when(s == NS - 1)
    def _():
        pltpu.sync_copy(acc.at[pl.ds(zbase, 400)],
                        out_hbm.at[c, pl.ds(zbase, 400)])


def _segment_sum_sc(ytab, gidx_cat, dst_pad, zeros):
    mesh = plsc.VectorSubcoreMesh(
        core_axis_name="c", subcore_axis_name="s", num_cores=NC,
        num_subcores=NS)
    f = functools.partial(
        pl.kernel,
        out_type=jax.ShapeDtypeStruct((2, N, HALF), jnp.float32),
        mesh=mesh,
        scratch_types=[
            pltpu.VMEM((CPS, CH), jnp.int32),
            pltpu.VMEM((CPS, CH), jnp.int32),
            pltpu.VMEM((2, CH, HALF), jnp.float32),
            pltpu.VMEM_SHARED((NACC, HALF), jnp.float32),
            pltpu.SemaphoreType.DMA,
            pltpu.SemaphoreType.DMA,
            pltpu.SemaphoreType.DMA,
        ],
    )(_scatter_body)
    return f(ytab, gidx_cat, dst_pad, zeros)


def _selfloop_body(x_ref, lw_ref, out_ref):
    out_ref[...] = jnp.dot(x_ref[...], lw_ref[...],
                           preferred_element_type=jnp.float32)


def _selfloop(x, loop_weight):
    return pl.pallas_call(
        _selfloop_body,
        grid=(N // TN_C,),
        in_specs=[
            pl.BlockSpec((TN_C, D), lambda n: (n, 0)),
            pl.BlockSpec((D, D), lambda n: (0, 0)),
        ],
        out_specs=pl.BlockSpec((TN_C, D), lambda n: (n, 0)),
        out_shape=jax.ShapeDtypeStruct((N, D), jnp.float32),
    )(x, loop_weight)


def _final_body(agg_ref, sl_ref, p_ref, out_ref):
    agg = jnp.concatenate([agg_ref[0], agg_ref[1]], axis=-1)
    mu = jnp.mean(agg, axis=-1, keepdims=True)
    var = jnp.mean((agg - mu) * (agg - mu), axis=-1, keepdims=True)
    h = (agg - mu) * lax.rsqrt(var + 1e-5) * p_ref[0] + p_ref[1]
    out_ref[...] = h + sl_ref[...]


def _finalize(agg2, selfloop, params):
    return pl.pallas_call(
        _final_body,
        grid=(N // TN_C,),
        in_specs=[
            pl.BlockSpec((2, TN_C, HALF), lambda n: (0, n, 0)),
            pl.BlockSpec((TN_C, D), lambda n: (n, 0)),
            pl.BlockSpec((2, D), lambda n: (0, 0)),
        ],
        out_specs=pl.BlockSpec((TN_C, D), lambda n: (n, 0)),
        out_shape=jax.ShapeDtypeStruct((N, D), jnp.float32),
    )(agg2, selfloop, params)


def kernel(x, edge_index, rel, W, loop_weight, h_bias, ln_gamma, ln_beta):
    src = edge_index[0]
    dst = edge_index[1]
    gidx = rel * N + src
    g0 = jnp.pad(gidx, (0, EPAD - E))
    gidx_cat = jnp.concatenate([g0, g0 + R * N])
    gidx_cat = gidx_cat.reshape(2 * EPAD // CH, CH)
    dst_pad = jnp.pad(dst, (0, EPAD - E), constant_values=N)
    dst_pad = dst_pad.reshape(EPAD // CH, CH)
    zeros = jnp.zeros((RPT, HALF), jnp.float32)
    params = jnp.stack([ln_gamma, ln_beta + h_bias])

    ytab = _build_ytab(x, W)
    agg2 = _segment_sum_sc(ytab, gidx_cat, dst_pad, zeros)
    selfloop = _selfloop(x, loop_weight)
    return _finalize(agg2, selfloop, params)


# speedup vs baseline: 1.2406x; 1.0152x over previous
import functools

import jax
import jax.numpy as jnp
from jax import lax
from jax.experimental import pallas as pl
from jax.experimental.pallas import tpu as pltpu
from jax.experimental.pallas import tpu_sc as plsc

N = 10000
D = 256
E = 160000
R = 16
B = 8
S = D // B
HALF = D // 2
NC = 2
NS = 16
CH = 128
EPT = 10240
STAGES = 2
CPS = EPT // (STAGES * CH)
EPAD = NS * EPT
NACC = N + 16
RPT = 640
TN_A = 10000
TN_C = 1000


def _ytab_body(x_ref, w_ref, out_ref, bdw):
    bdw[...] = jnp.zeros((D, D), jnp.float32)
    for b in range(B):
        bdw[b * S:(b + 1) * S, b * S:(b + 1) * S] = w_ref[0, b]
    y = jnp.dot(x_ref[...], bdw[...], preferred_element_type=jnp.float32)
    out_ref[0, 0] = y[:, :HALF]
    out_ref[1, 0] = y[:, HALF:]


def _build_ytab(x, W):
    out = pl.pallas_call(
        _ytab_body,
        grid=(N // TN_A, R),
        in_specs=[
            pl.BlockSpec((TN_A, D), lambda n, r: (n, 0)),
            pl.BlockSpec((1, B, S, S), lambda n, r: (r, 0, 0, 0)),
        ],
        out_specs=pl.BlockSpec((2, 1, TN_A, HALF), lambda n, r: (0, r, n, 0)),
        out_shape=jax.ShapeDtypeStruct((2, R, N, HALF), jnp.float32),
        scratch_shapes=[pltpu.VMEM((D, D), jnp.float32)],
    )(x, W)
    return out.reshape(2 * R * N, HALF)


def _scatter_body(ytab_hbm, gidx_hbm, dst_hbm, zeros_hbm, out_hbm,
                  gi_v, di_v, rows_v, acc, sg0, sg1, ss):
    sg = (sg0, sg1)
    c = lax.axis_index("c")
    s = lax.axis_index("s")
    zbase = pl.multiple_of(s * RPT, 128)

    @pl.when(s < NS - 1)
    def _():
        pltpu.sync_copy(zeros_hbm, acc.at[pl.ds(zbase, RPT)])

    @pl.when(s == NS - 1)
    def _():
        pltpu.sync_copy(zeros_hbm.at[pl.ds(0, 400)],
                        acc.at[pl.ds(zbase, 400)])

    plsc.subcore_barrier()
    grow = c * (EPAD // CH) + s * (EPT // CH)
    drow = s * (EPT // CH)

    for h in range(STAGES):
        goff = pl.multiple_of(grow + h * CPS, 8)
        doff = pl.multiple_of(drow + h * CPS, 8)
        pltpu.sync_copy(gidx_hbm.at[pl.ds(goff, CPS)], gi_v)
        pltpu.sync_copy(dst_hbm.at[pl.ds(doff, CPS)], di_v)
        pltpu.async_copy(ytab_hbm.at[gi_v.at[0]], rows_v.at[0], sg[0])
        pltpu.async_copy(ytab_hbm.at[gi_v.at[1]], rows_v.at[1], sg[1])

        def super_round(j, carry):
            for b in range(2):
                k = 2 * j + b
                pltpu.make_async_copy(ytab_hbm.at[gi_v.at[k]],
                                      rows_v.at[b], sg[b]).wait()
                pltpu.async_copy(rows_v.at[b], acc.at[di_v.at[k]], ss,
                                 add=True).wait()

                @pl.when(j < CPS // 2 - 1)
                def _():
                    pltpu.async_copy(ytab_hbm.at[gi_v.at[k + 2]],
                                     rows_v.at[b], sg[b])
            return carry

        lax.fori_loop(0, CPS // 2, super_round, 0)
    plsc.subcore_barrier()

    @pl.when(s < NS - 1)
    def _():
        pltpu.sync_copy(acc.at[pl.ds(zbase, RPT)],
                        out_hbm.at[c, pl.ds(zbase, RPT)])

    @pl.when(s == NS - 1)
    def _():
        pltpu.sync_copy(acc.at[pl.ds(zbase, 400)],
                        out_hbm.at[c, pl.ds(zbase, 400)])


def _segment_sum_sc(ytab, gidx_cat, dst_pad, zeros):
    mesh = plsc.VectorSubcoreMesh(
        core_axis_name="c", subcore_axis_name="s", num_cores=NC,
        num_subcores=NS)
    f = functools.partial(
        pl.kernel,
        out_type=jax.ShapeDtypeStruct((2, N, HALF), jnp.float32),
        mesh=mesh,
        scratch_types=[
            pltpu.VMEM((CPS, CH), jnp.int32),
            pltpu.VMEM((CPS, CH), jnp.int32),
            pltpu.VMEM((2, CH, HALF), jnp.float32),
            pltpu.VMEM_SHARED((NACC, HALF), jnp.float32),
            pltpu.SemaphoreType.DMA,
            pltpu.SemaphoreType.DMA,
            pltpu.SemaphoreType.DMA,
        ],
    )(_scatter_body)
    return f(ytab, gidx_cat, dst_pad, zeros)


def _selfloop_body(x_ref, lw_ref, out_ref):
    out_ref[...] = jnp.dot(x_ref[...], lw_ref[...],
                           preferred_element_type=jnp.float32)


def _selfloop(x, loop_weight):
    return pl.pallas_call(
        _selfloop_body,
        grid=(N // TN_C,),
        in_specs=[
            pl.BlockSpec((TN_C, D), lambda n: (n, 0)),
            pl.BlockSpec((D, D), lambda n: (0, 0)),
        ],
        out_specs=pl.BlockSpec((TN_C, D), lambda n: (n, 0)),
        out_shape=jax.ShapeDtypeStruct((N, D), jnp.float32),
    )(x, loop_weight)


def _final_body(agg_ref, sl_ref, p_ref, out_ref):
    agg = jnp.concatenate([agg_ref[0], agg_ref[1]], axis=-1)
    mu = jnp.mean(agg, axis=-1, keepdims=True)
    var = jnp.mean((agg - mu) * (agg - mu), axis=-1, keepdims=True)
    h = (agg - mu) * lax.rsqrt(var + 1e-5) * p_ref[0] + p_ref[1]
    out_ref[...] = h + sl_ref[...]


def _finalize(agg2, selfloop, params):
    return pl.pallas_call(
        _final_body,
        grid=(N // TN_C,),
        in_specs=[
            pl.BlockSpec((2, TN_C, HALF), lambda n: (0, n, 0)),
            pl.BlockSpec((TN_C, D), lambda n: (n, 0)),
            pl.BlockSpec((2, D), lambda n: (0, 0)),
        ],
        out_specs=pl.BlockSpec((TN_C, D), lambda n: (n, 0)),
        out_shape=jax.ShapeDtypeStruct((N, D), jnp.float32),
    )(agg2, selfloop, params)


def kernel(x, edge_index, rel, W, loop_weight, h_bias, ln_gamma, ln_beta):
    src = edge_index[0]
    dst = edge_index[1]
    gidx = rel * N + src
    g0 = jnp.pad(gidx, (0, EPAD - E))
    gidx_cat = jnp.concatenate([g0, g0 + R * N])
    gidx_cat = gidx_cat.reshape(2 * EPAD // CH, CH)
    dst_pad = jnp.pad(dst, (0, EPAD - E), constant_values=N)
    dst_pad = dst_pad.reshape(EPAD // CH, CH)
    zeros = jnp.zeros((RPT, HALF), jnp.float32)
    params = jnp.stack([ln_gamma, ln_beta + h_bias])

    ytab = _build_ytab(x, W)
    agg2 = _segment_sum_sc(ytab, gidx_cat, dst_pad, zeros)
    selfloop = _selfloop(x, loop_weight)
    return _finalize(agg2, selfloop, params)
